# baseline (speedup 1.0000x reference)
import functools

import jax
import jax.numpy as jnp
from jax import lax
from jax.experimental import pallas as pl
from jax.experimental.pallas import tpu as pltpu
from jax.experimental.pallas import tpu_sc as plsc

NC = 2
NS = 16
NW = NC * NS

D = 128
CHUNK = 128
NBUF = 5
LAG = 2


@functools.partial(jax.jit, static_argnames=("b_per_w", "n_chunks"))
def _embed_lookup(idx_grp, table, *, b_per_w, n_chunks):
    B = NW * b_per_w
    assert (n_chunks - NBUF) % NBUF == 0

    mesh = plsc.VectorSubcoreMesh(
        core_axis_name="c", subcore_axis_name="s", num_cores=NC, num_subcores=NS
    )

    @functools.partial(
        pl.kernel,
        out_type=jax.ShapeDtypeStruct((B, D), jnp.float32),
        mesh=mesh,
        scratch_types=[
            pltpu.VMEM((n_chunks, CHUNK), jnp.int32),
            pltpu.VMEM((NBUF, CHUNK, D), jnp.float32),
            pltpu.SemaphoreType.DMA((NBUF,)),
            pltpu.SemaphoreType.DMA((NBUF,)),
        ],
    )
    def body(idx_hbm, table_hbm, out_hbm, idx_v, rows_v, gsem, ssem):
        wid = lax.axis_index("s") * NC + lax.axis_index("c")
        base = wid * b_per_w
        pltpu.sync_copy(idx_hbm.at[wid], idx_v)

        def gather(g, b):
            return pltpu.make_async_copy(
                table_hbm.at[idx_v.at[g]], rows_v.at[b], gsem.at[b]
            )

        def store(g, b):
            return pltpu.make_async_copy(
                rows_v.at[b], out_hbm.at[pl.ds(base + g * CHUNK, CHUNK)], ssem.at[b]
            )

        def step(g, b, do_swait, do_complete):
            if do_complete:
                b2 = (b - LAG) % NBUF
                gather(g - LAG, b2).wait()
                store(g - LAG, b2).start()
            if do_swait:
                store(g - NBUF, b).wait()
            gather(g, b).start()

        for g in range(NBUF):
            step(g, g % NBUF, do_swait=False, do_complete=g >= LAG)

        def outer(it, carry):
            g0 = NBUF + it * NBUF
            for b0 in range(NBUF):
                step(g0 + b0, b0, do_swait=True, do_complete=True)
            return carry

        lax.fori_loop(0, (n_chunks - NBUF) // NBUF, outer, 0)

        for g in range(n_chunks, n_chunks + LAG):
            b2 = (g - LAG) % NBUF
            gather(g - LAG, b2).wait()
            store(g - LAG, b2).start()
        for g in range(n_chunks - NBUF, n_chunks):
            store(g, g % NBUF).wait()

    return body(idx_grp, table)


def kernel(input_ids, table):
    Bt, T = input_ids.shape
    B = Bt * T
    assert B % (NW * CHUNK) == 0
    b_per_w = B // NW
    n_chunks = b_per_w // CHUNK
    idx_grp = input_ids.reshape(NW, n_chunks, CHUNK).astype(jnp.int32)
    out = _embed_lookup(idx_grp, table, b_per_w=b_per_w, n_chunks=n_chunks)
    return out.reshape(Bt, T, D)


# speedup vs baseline: 1.0004x; 1.0004x over previous
import functools

import jax
import jax.numpy as jnp
from jax import lax
from jax.experimental import pallas as pl
from jax.experimental.pallas import tpu as pltpu
from jax.experimental.pallas import tpu_sc as plsc

NC = 2
NS = 16
NW = NC * NS

D = 128
CHUNK = 128
NBUF = 6
LAG = 3


@functools.partial(jax.jit, static_argnames=("b_per_w", "n_chunks"))
def _embed_lookup(idx_grp, table, *, b_per_w, n_chunks):
    B = NW * b_per_w

    mesh = plsc.VectorSubcoreMesh(
        core_axis_name="c", subcore_axis_name="s", num_cores=NC, num_subcores=NS
    )

    @functools.partial(
        pl.kernel,
        out_type=jax.ShapeDtypeStruct((B, D), jnp.float32),
        mesh=mesh,
        scratch_types=[
            pltpu.VMEM((n_chunks, CHUNK), jnp.int32),
            pltpu.VMEM((NBUF, CHUNK, D), jnp.float32),
            pltpu.SemaphoreType.DMA((NBUF,)),
            pltpu.SemaphoreType.DMA((NBUF,)),
        ],
    )
    def body(idx_hbm, table_hbm, out_hbm, idx_v, rows_v, gsem, ssem):
        wid = lax.axis_index("s") * NC + lax.axis_index("c")
        base = wid * b_per_w
        pltpu.sync_copy(idx_hbm.at[wid], idx_v)

        def gather(g, b):
            return pltpu.make_async_copy(
                table_hbm.at[idx_v.at[g]], rows_v.at[b], gsem.at[b]
            )

        def store(g, b):
            return pltpu.make_async_copy(
                rows_v.at[b], out_hbm.at[pl.ds(base + g * CHUNK, CHUNK)], ssem.at[b]
            )

        def step(g, b, do_swait, do_complete):
            if do_complete:
                b2 = (b - LAG) % NBUF
                gather(g - LAG, b2).wait()
                store(g - LAG, b2).start()
            if do_swait:
                store(g - NBUF, b).wait()
            gather(g, b).start()

        for g in range(NBUF):
            step(g, g % NBUF, do_swait=False, do_complete=g >= LAG)

        n_main = (n_chunks - NBUF) // NBUF

        def outer(it, carry):
            g0 = NBUF + it * NBUF
            for b0 in range(NBUF):
                step(g0 + b0, b0, do_swait=True, do_complete=True)
            return carry

        lax.fori_loop(0, n_main, outer, 0)
        for g in range(NBUF + n_main * NBUF, n_chunks):
            step(g, g % NBUF, do_swait=True, do_complete=True)

        for g in range(n_chunks, n_chunks + LAG):
            b2 = (g - LAG) % NBUF
            gather(g - LAG, b2).wait()
            store(g - LAG, b2).start()
        for g in range(n_chunks - NBUF, n_chunks):
            store(g, g % NBUF).wait()

    return body(idx_grp, table)


def kernel(input_ids, table):
    Bt, T = input_ids.shape
    B = Bt * T
    assert B % (NW * CHUNK) == 0
    b_per_w = B // NW
    n_chunks = b_per_w // CHUNK
    idx_grp = input_ids.reshape(NW, n_chunks, CHUNK).astype(jnp.int32)
    out = _embed_lookup(idx_grp, table, b_per_w=b_per_w, n_chunks=n_chunks)
    return out.reshape(Bt, T, D)
